# baseline (speedup 1.0000x reference)
import functools

import jax
import jax.numpy as jnp
from jax import lax
from jax.experimental import pallas as pl
from jax.experimental.pallas import tpu as pltpu
from jax.experimental.pallas import tpu_sc as plsc

B, T, D = 4096, 200, 128
N = B * T
NC, NS, L = 2, 16, 16
NW = NC * NS
ROWS_PER_W = N // NW
BB = B // NW
NTB = T // 8
CH = 64
NG = CH // L
NCHUNK = ROWS_PER_W // CH


def _make_sc_lookup():
    mesh = plsc.VectorSubcoreMesh(core_axis_name="c", subcore_axis_name="s",
                                  num_cores=NC, num_subcores=NS)

    @functools.partial(
        pl.kernel,
        mesh=mesh,
        compiler_params=pltpu.CompilerParams(use_tc_tiling_on_sc=True,
                                             needs_layout_passes=False),
        out_type=jax.ShapeDtypeStruct((N, 3 * D), jnp.float32),
        scratch_types=[
            pltpu.VMEM((8, 128), jnp.int32),
            pltpu.VMEM((8, 128), jnp.int32),
            pltpu.VMEM((8, 128), jnp.int32),
            pltpu.VMEM((8, 128), jnp.float32),
            pltpu.VMEM((8, 128), jnp.float32),
            pltpu.VMEM((8, 128), jnp.float32),
            pltpu.VMEM((ROWS_PER_W,), jnp.int32),
            pltpu.VMEM((CH, 3 * D), jnp.float32),
            pltpu.VMEM((CH, 3 * D), jnp.float32),
            pltpu.SemaphoreType.DMA,
            pltpu.SemaphoreType.DMA,
            pltpu.SemaphoreType.DMA,
        ],
    )
    def sc_lookup(idx_hbm, w1_hbm, w2_hbm, w3_hbm, out_hbm,
                  t1v, t2v, t3v, w1v, w2v, w3v, cbuf,
                  rows0, rows1, isem, ssem0, ssem1):
        wid = lax.axis_index("s") * NC + lax.axis_index("c")
        lanes = jnp.arange(L, dtype=jnp.int32)
        lanes_t = lanes * T
        rows = (rows0, rows1)
        ssem = (ssem0, ssem1)
        b0 = wid * BB
        row_base = wid * ROWS_PER_W

        pltpu.sync_copy(w1_hbm.at[pl.ds(0, 8)], w1v)
        pltpu.sync_copy(w2_hbm.at[pl.ds(0, 8)], w2v)
        pltpu.sync_copy(w3_hbm.at[pl.ds(0, 8)], w3v)

        def fuse_tile(tb, carry):
            cps = [
                pltpu.async_copy(
                    idx_hbm.at[ch, pl.ds(tb * 8, 8), pl.ds(b0, BB)],
                    tv, isem)
                for ch, tv in ((0, t1v), (1, t2v), (2, t3v))
            ]
            for cp in cps:
                cp.wait()
            t8 = tb * 8
            for ti in range(8):
                for j in range(8):
                    s = pl.ds(j * L, L)
                    c = t1v[ti, s] * 64 + t2v[ti, s] * 8 + t3v[ti, s]
                    addr = lanes_t + (j * L * T + t8 + ti)
                    plsc.store_scatter(cbuf, [addr], c)
            return carry

        lax.fori_loop(0, NTB, fuse_tile, 0, unroll=False)

        def fire_scatter(g, b):
            pltpu.async_copy(rows[b],
                             out_hbm.at[pl.ds(row_base + g * CH, CH)], ssem[b])

        def wait_scatter(g, b):
            pltpu.make_async_copy(
                rows[b], out_hbm.at[pl.ds(row_base + g * CH, CH)],
                ssem[b]).wait()

        def chunk_body(g, carry):
            for b in range(2):
                gg = 2 * g + b

                @pl.when(gg >= 2)
                def _():
                    wait_scatter(gg - 2, b)
                a_idx = []
                r_idx = []
                for q in range(NG):
                    cv = cbuf[pl.ds(gg * CH + q * L, L)]
                    a_idx.append((cv >> 6, (cv >> 3) & 7, cv & 7))
                    r_idx.append(lanes + q * L)
                wvs = (w1v, w2v, w3v)

                def col_body(i, carry):
                    for u in range(4):
                        jl = i * 4 + u
                        wcol = jnp.full((L,), 1, dtype=jnp.int32) * jl
                        ocols = (wcol, wcol + 128, wcol + 256)
                        for ch in range(3):
                            for q in range(NG):
                                v = plsc.load_gather(wvs[ch],
                                                     [a_idx[q][ch], wcol])
                                plsc.store_scatter(rows[b],
                                                   [r_idx[q], ocols[ch]], v)
                    return carry

                lax.fori_loop(0, 32, col_body, 0, unroll=False)
                fire_scatter(gg, b)
            return carry

        lax.fori_loop(0, NCHUNK // 2, chunk_body, 0, unroll=False)
        wait_scatter(NCHUNK - 2, 0)
        wait_scatter(NCHUNK - 1, 1)

    return sc_lookup


_sc_lookup = _make_sc_lookup()


def kernel(attr_seq_tsr, W1, W2, W3):
    idx_t = attr_seq_tsr.astype(jnp.int32).transpose(2, 1, 0)
    out = _sc_lookup(idx_t, W1, W2, W3)
    return out.reshape(B, T, 3 * D)


# speedup vs baseline: 9.5620x; 9.5620x over previous
import functools

import jax
import jax.numpy as jnp
from jax import lax
from jax.experimental import pallas as pl
from jax.experimental.pallas import tpu as pltpu
from jax.experimental.pallas import tpu_sc as plsc

B, T, D = 4096, 200, 128
N = B * T
NC, NS, L = 2, 16, 16
NW = NC * NS
ROWS_PER_W = N // NW
BB = B // NW
NTB = T // 8
CH = 80
NB = 3
NCHUNK = ROWS_PER_W // CH


def _make_sc_lookup():
    mesh = plsc.VectorSubcoreMesh(core_axis_name="c", subcore_axis_name="s",
                                  num_cores=NC, num_subcores=NS)

    @functools.partial(
        pl.kernel,
        mesh=mesh,
        compiler_params=pltpu.CompilerParams(use_tc_tiling_on_sc=True,
                                             needs_layout_passes=False),
        out_type=jax.ShapeDtypeStruct((N, 3 * D), jnp.float32),
        scratch_types=[
            pltpu.VMEM((8, 128), jnp.int32),
            pltpu.VMEM((8, 128), jnp.int32),
            pltpu.VMEM((8, 128), jnp.int32),
            pltpu.VMEM((ROWS_PER_W,), jnp.int32),
            pltpu.VMEM((CH, 3 * D), jnp.float32),
            pltpu.VMEM((CH, 3 * D), jnp.float32),
            pltpu.VMEM((CH, 3 * D), jnp.float32),
            pltpu.SemaphoreType.DMA,
            pltpu.SemaphoreType.DMA,
            pltpu.SemaphoreType.DMA,
            pltpu.SemaphoreType.DMA,
            pltpu.SemaphoreType.DMA,
            pltpu.SemaphoreType.DMA,
            pltpu.SemaphoreType.DMA,
        ],
    )
    def sc_lookup(tc_hbm, idx_hbm, out_hbm, t1v, t2v, t3v, cbuf,
                  rows0, rows1, rows2, isem, gsem0, gsem1, gsem2,
                  ssem0, ssem1, ssem2):
        wid = lax.axis_index("s") * NC + lax.axis_index("c")
        lanes = jnp.arange(L, dtype=jnp.int32)
        lanes_t = lanes * T
        rows = (rows0, rows1, rows2)
        gsem = (gsem0, gsem1, gsem2)
        ssem = (ssem0, ssem1, ssem2)
        b0 = wid * BB
        row_base = wid * ROWS_PER_W

        def fuse_tile(tb, carry):
            cps = [
                pltpu.async_copy(
                    idx_hbm.at[ch, pl.ds(tb * 8, 8), pl.ds(b0, BB)],
                    tv, isem)
                for ch, tv in ((0, t1v), (1, t2v), (2, t3v))
            ]
            for cp in cps:
                cp.wait()
            t8 = tb * 8
            for ti in range(8):
                for j in range(8):
                    s = pl.ds(j * L, L)
                    c = t1v[ti, s] * 64 + t2v[ti, s] * 8 + t3v[ti, s]
                    addr = lanes_t + (j * L * T + t8 + ti)
                    plsc.store_scatter(cbuf, [addr], c)
            return carry

        lax.fori_loop(0, NTB, fuse_tile, 0, unroll=False)

        def fire_gather(g, b):
            pltpu.async_copy(tc_hbm.at[cbuf.at[pl.ds(g * CH, CH)]],
                             rows[b], gsem[b])

        def wait_gather(g, b):
            pltpu.make_async_copy(tc_hbm.at[cbuf.at[pl.ds(g * CH, CH)]],
                                  rows[b], gsem[b]).wait()

        def fire_scatter(g, b):
            pltpu.async_copy(rows[b],
                             out_hbm.at[pl.ds(row_base + g * CH, CH)], ssem[b])

        def wait_scatter(g, b):
            pltpu.make_async_copy(
                rows[b], out_hbm.at[pl.ds(row_base + g * CH, CH)],
                ssem[b]).wait()

        fire_gather(0, 0)
        fire_gather(1, 1)

        def tri_body(p, carry):
            for b in range(NB):
                g = NB * p + b
                nb = (b + 2) % NB

                @pl.when(g + 2 < NCHUNK)
                def _():
                    @pl.when(g >= 1)
                    def _():
                        wait_scatter(g - 1, nb)
                    fire_gather(g + 2, nb)
                wait_gather(g, b)
                fire_scatter(g, b)
            return carry

        lax.fori_loop(0, NCHUNK // NB, tri_body, 0, unroll=False)
        for g in range(NB * (NCHUNK // NB), NCHUNK):
            wait_gather(g, g % NB)
            fire_scatter(g, g % NB)
        for g in range(NCHUNK - NB, NCHUNK):
            wait_scatter(g, g % NB)

    return sc_lookup


_sc_lookup = _make_sc_lookup()


def kernel(attr_seq_tsr, W1, W2, W3):
    p1 = jnp.broadcast_to(W1[:8, None, None, :], (8, 8, 8, D))
    p2 = jnp.broadcast_to(W2[None, :8, None, :], (8, 8, 8, D))
    p3 = jnp.broadcast_to(W3[None, None, :8, :], (8, 8, 8, D))
    tc = jnp.concatenate([p1, p2, p3], axis=-1).reshape(512, 3 * D)
    idx_t = attr_seq_tsr.astype(jnp.int32).transpose(2, 1, 0)
    out = _sc_lookup(tc, idx_t)
    return out.reshape(B, T, 3 * D)
